# baseline (speedup 1.0000x reference)
import jax
import jax.numpy as jnp
from jax.experimental import pallas as pl
from jax.experimental.pallas import tpu as pltpu

_E = 64
_EP = 128
_K = 8


def _router_block(x_ref, wt_ref, topk_p_ref, topk_i_ref, probs_ref):
    x = x_ref[...]
    wt = wt_ref[...]
    logits = jnp.dot(x, wt, preferred_element_type=jnp.float32)

    iota_f = jax.lax.broadcasted_iota(
        jnp.int32, logits.shape, 1).astype(jnp.float32)
    lg = jnp.where(iota_f < _E, logits, -jnp.inf)
    m = jnp.max(lg, axis=-1, keepdims=True)
    e = jnp.exp(lg - m)
    s = jnp.sum(e, axis=-1, keepdims=True)
    p = e / s
    probs_ref[...] = p[:, :_E]

    work = p
    vals = []
    idxs = []
    for _ in range(_K):
        mx = jnp.max(work, axis=-1, keepdims=True)
        ixf = jnp.min(jnp.where(work == mx, iota_f, float(_EP)),
                      axis=-1, keepdims=True)
        work = jnp.where(iota_f == ixf, -1.0, work)
        vals.append(mx)
        idxs.append(ixf)
    tv = jnp.concatenate(vals, axis=-1)
    ti = jnp.concatenate(idxs, axis=-1)
    denom = jnp.sum(tv, axis=-1, keepdims=True) + 1e-9
    topk_p_ref[...] = tv / denom
    topk_i_ref[...] = ti.astype(jnp.int32)


def kernel(x, W):
    B, S, D = x.shape
    N = B * S
    xf = x.reshape(N, D)
    wt = jnp.zeros((D, _EP), jnp.float32).at[:, :_E].set(W.T)

    TB = 1024
    grid = (N // TB,)
    tp, ti, ap = pl.pallas_call(
        _router_block,
        grid=grid,
        in_specs=[
            pl.BlockSpec((TB, D), lambda i: (i, 0)),
            pl.BlockSpec((D, _EP), lambda i: (0, 0)),
        ],
        out_specs=[
            pl.BlockSpec((TB, _K), lambda i: (i, 0)),
            pl.BlockSpec((TB, _K), lambda i: (i, 0)),
            pl.BlockSpec((TB, _E), lambda i: (i, 0)),
        ],
        out_shape=[
            jax.ShapeDtypeStruct((N, _K), jnp.float32),
            jax.ShapeDtypeStruct((N, _K), jnp.int32),
            jax.ShapeDtypeStruct((N, _E), jnp.float32),
        ],
        compiler_params=pltpu.CompilerParams(
            dimension_semantics=("arbitrary",),
        ),
    )(xf, wt)
    return (tp.reshape(B, S, _K), ti.reshape(B, S, _K), ap.reshape(B, S, _E))


# speedup vs baseline: 1.0327x; 1.0327x over previous
import jax
import jax.numpy as jnp
from jax.experimental import pallas as pl
from jax.experimental.pallas import tpu as pltpu

_E = 64
_EP = 128
_K = 8


def _router_block(x_ref, wt_ref, topk_p_ref, topk_i_ref, probs_ref):
    x = x_ref[...]
    wt = wt_ref[...]
    logits = jnp.dot(x, wt, preferred_element_type=jnp.float32)

    iota_f = jax.lax.broadcasted_iota(
        jnp.int32, logits.shape, 1).astype(jnp.float32)
    lg = jnp.where(iota_f < _E, logits, -jnp.inf)
    m = jnp.max(lg, axis=-1, keepdims=True)
    e = jnp.exp(lg - m)
    s = jnp.sum(e, axis=-1, keepdims=True)
    p = e / s
    probs_ref[...] = p[:, :_E]

    work = p
    vals = []
    idxs = []
    for _ in range(_K):
        mx = jnp.max(work, axis=-1, keepdims=True)
        ixf = jnp.min(jnp.where(work == mx, iota_f, float(_EP)),
                      axis=-1, keepdims=True)
        work = jnp.where(iota_f == ixf, -1.0, work)
        vals.append(mx)
        idxs.append(ixf)
    tv = jnp.concatenate(vals, axis=-1)
    ti = jnp.concatenate(idxs, axis=-1)
    denom = jnp.sum(tv, axis=-1, keepdims=True) + 1e-9
    topk_p_ref[...] = tv / denom
    topk_i_ref[...] = ti.astype(jnp.int32)


def kernel(x, W):
    B, S, D = x.shape
    N = B * S
    xf = x.reshape(N, D)
    wt = jnp.zeros((D, _EP), jnp.float32).at[:, :_E].set(W.T)

    TB = 4096
    grid = (N // TB,)
    tp, ti, ap = pl.pallas_call(
        _router_block,
        grid=grid,
        in_specs=[
            pl.BlockSpec((TB, D), lambda i: (i, 0)),
            pl.BlockSpec((D, _EP), lambda i: (0, 0)),
        ],
        out_specs=[
            pl.BlockSpec((TB, _K), lambda i: (i, 0)),
            pl.BlockSpec((TB, _K), lambda i: (i, 0)),
            pl.BlockSpec((TB, _E), lambda i: (i, 0)),
        ],
        out_shape=[
            jax.ShapeDtypeStruct((N, _K), jnp.float32),
            jax.ShapeDtypeStruct((N, _K), jnp.int32),
            jax.ShapeDtypeStruct((N, _E), jnp.float32),
        ],
        compiler_params=pltpu.CompilerParams(
            dimension_semantics=("arbitrary",),
        ),
    )(xf, wt)
    return (tp.reshape(B, S, _K), ti.reshape(B, S, _K), ap.reshape(B, S, _E))


# speedup vs baseline: 1.5204x; 1.4722x over previous
import jax
import jax.numpy as jnp
from jax.experimental import pallas as pl
from jax.experimental.pallas import tpu as pltpu

_E = 64
_K = 8


def _router_block(x_ref, w_ref, topk_p_ref, topk_i_ref, probs_ref):
    x = x_ref[...]
    w = w_ref[...]
    lgT = jax.lax.dot_general(
        w, x, (((1,), (1,)), ((), ())),
        preferred_element_type=jnp.float32)

    iota_s = jax.lax.broadcasted_iota(
        jnp.int32, lgT.shape, 0).astype(jnp.float32)
    m = jnp.max(lgT, axis=0, keepdims=True)
    e = jnp.exp(lgT - m)
    s = jnp.sum(e, axis=0, keepdims=True)
    p = e * (1.0 / s)
    probs_ref[...] = p.T

    work = p
    vals = []
    idxs = []
    for _ in range(_K):
        mx = jnp.max(work, axis=0, keepdims=True)
        ixf = jnp.min(jnp.where(work == mx, iota_s, float(_E)),
                      axis=0, keepdims=True)
        work = jnp.where(iota_s == ixf, -1.0, work)
        vals.append(mx)
        idxs.append(ixf)
    tv = jnp.concatenate(vals, axis=0)
    ti = jnp.concatenate(idxs, axis=0)
    denom = jnp.sum(tv, axis=0, keepdims=True) + 1e-9
    topk_p_ref[...] = (tv / denom).T
    topk_i_ref[...] = ti.astype(jnp.int32).T


def kernel(x, W):
    B, S, D = x.shape
    N = B * S
    xf = x.reshape(N, D)

    TB = 2048
    grid = (N // TB,)
    tp, ti, ap = pl.pallas_call(
        _router_block,
        grid=grid,
        in_specs=[
            pl.BlockSpec((TB, D), lambda i: (i, 0)),
            pl.BlockSpec((_E, D), lambda i: (0, 0)),
        ],
        out_specs=[
            pl.BlockSpec((TB, _K), lambda i: (i, 0)),
            pl.BlockSpec((TB, _K), lambda i: (i, 0)),
            pl.BlockSpec((TB, _E), lambda i: (i, 0)),
        ],
        out_shape=[
            jax.ShapeDtypeStruct((N, _K), jnp.float32),
            jax.ShapeDtypeStruct((N, _K), jnp.int32),
            jax.ShapeDtypeStruct((N, _E), jnp.float32),
        ],
        compiler_params=pltpu.CompilerParams(
            dimension_semantics=("arbitrary",),
        ),
    )(xf, W)
    return (tp.reshape(B, S, _K), ti.reshape(B, S, _K), ap.reshape(B, S, _E))


# speedup vs baseline: 1.5774x; 1.0375x over previous
import jax
import jax.numpy as jnp
from jax.experimental import pallas as pl
from jax.experimental.pallas import tpu as pltpu

_E = 64
_K = 8


def _router_block(x_ref, w_ref, topk_p_ref, topk_i_ref, probs_ref):
    x = x_ref[...]
    w = w_ref[...]
    lgT = jax.lax.dot_general(
        w, x, (((1,), (1,)), ((), ())),
        preferred_element_type=jnp.float32)

    iota_s = jax.lax.broadcasted_iota(
        jnp.int32, lgT.shape, 0).astype(jnp.float32)
    m = jnp.max(lgT, axis=0, keepdims=True)
    e = jnp.exp(lgT - m)
    s = jnp.sum(e, axis=0, keepdims=True)
    p = e * (1.0 / s)
    probs_ref[...] = p.T

    work = p
    vals = []
    idxs = []
    for _ in range(_K):
        mx = jnp.max(work, axis=0, keepdims=True)
        ixf = jnp.min(jnp.where(work == mx, iota_s, float(_E)),
                      axis=0, keepdims=True)
        work = jnp.where(iota_s == ixf, -1.0, work)
        vals.append(mx)
        idxs.append(ixf)
    tv = jnp.concatenate(vals, axis=0)
    ti = jnp.concatenate(idxs, axis=0)
    denom = jnp.sum(tv, axis=0, keepdims=True) + 1e-9
    topk_p_ref[...] = (tv / denom).T
    topk_i_ref[...] = ti.astype(jnp.int32).T


def kernel(x, W):
    B, S, D = x.shape
    N = B * S
    xf = x.reshape(N, D)

    TB = 4096
    grid = (N // TB,)
    tp, ti, ap = pl.pallas_call(
        _router_block,
        grid=grid,
        in_specs=[
            pl.BlockSpec((TB, D), lambda i: (i, 0)),
            pl.BlockSpec((_E, D), lambda i: (0, 0)),
        ],
        out_specs=[
            pl.BlockSpec((TB, _K), lambda i: (i, 0)),
            pl.BlockSpec((TB, _K), lambda i: (i, 0)),
            pl.BlockSpec((TB, _E), lambda i: (i, 0)),
        ],
        out_shape=[
            jax.ShapeDtypeStruct((N, _K), jnp.float32),
            jax.ShapeDtypeStruct((N, _K), jnp.int32),
            jax.ShapeDtypeStruct((N, _E), jnp.float32),
        ],
        compiler_params=pltpu.CompilerParams(
            dimension_semantics=("arbitrary",),
        ),
    )(xf, W)
    return (tp.reshape(B, S, _K), ti.reshape(B, S, _K), ap.reshape(B, S, _E))
